# baseline (speedup 1.0000x reference)
import functools

import jax
import jax.numpy as jnp
from jax.experimental import pallas as pl
from jax.experimental.pallas import tpu as pltpu
from jax.experimental.pallas import tpu_sc as plsc

B, S, H = 2, 2048, 2048
NH = 16
DH = H // NH
P = 16
K = 4
RH = 256
I = 8192
HPP = H // P
IPP = I // P
M = B * S

BM_QKV = 512
BQ = 512
BK = 512
BM_EPI = 512

_BF = jnp.bfloat16
_F32 = jnp.float32


def _dot_nt(a, b_t):
    return jax.lax.dot_general(
        a, b_t, (((1,), (1,)), ((), ())), preferred_element_type=_F32)


def _layer_norm(x, s, b, eps=1e-5):
    m = jnp.mean(x, axis=-1, keepdims=True)
    v = jnp.mean((x - m) ** 2, axis=-1, keepdims=True)
    return (x - m) * jax.lax.rsqrt(v + eps) * s + b


def _qkv_kernel(x_ref, s_ref, b_ref, w_ref, bias_ref, o_ref):
    h = _layer_norm(x_ref[...], s_ref[...], b_ref[...])
    acc = _dot_nt(h.astype(_BF), w_ref[...])
    o_ref[...] = (acc + bias_ref[...]).astype(_BF)


def _attn_kernel(q_ref, k_ref, v_ref, o_ref, acc_ref, m_ref, l_ref):
    qi = pl.program_id(2)
    q = q_ref[0]
    iq = jax.lax.broadcasted_iota(jnp.int32, (BQ, BK), 0)
    ik = jax.lax.broadcasted_iota(jnp.int32, (BQ, BK), 1)

    for j in range(S // BK):
        @pl.when(j <= qi)
        def _chunk(j=j):
            k = k_ref[0, j * BK:(j + 1) * BK, :]
            s = jax.lax.dot_general(
                q, k, (((1,), (1,)), ((), ())), preferred_element_type=_F32)
            s = jnp.where((qi - j) * BK + iq >= ik, s, -1e30)
            mc = jnp.max(s, axis=1, keepdims=True)
            if j == 0:
                m_new = mc
            else:
                m_new = jnp.maximum(m_ref[...], mc)
            p = jnp.exp(s - m_new)
            ps = jnp.sum(p, axis=1, keepdims=True)
            pv = jnp.dot(p.astype(_BF), v_ref[0, j * BK:(j + 1) * BK, :],
                         preferred_element_type=_F32)
            if j == 0:
                l_ref[...] = ps
                acc_ref[...] = pv
            else:
                alpha = jnp.exp(m_ref[...] - m_new)
                l_ref[...] = l_ref[...] * alpha + ps
                acc_ref[...] = acc_ref[...] * alpha + pv
            m_ref[...] = m_new

    o_ref[0] = (acc_ref[...] / l_ref[...]).astype(_BF)


def _epi_a_kernel(res_ref, o_ref, wo_ref, bo_ref, s2_ref, b2_ref,
                  wr1_ref, br1_ref, wr2_ref, br2_ref,
                  x_ref, h2_ref, sc_ref):
    attn = _dot_nt(o_ref[...], wo_ref[...])
    x = res_ref[...] + attn + bo_ref[...]
    x_ref[...] = x
    h2 = _layer_norm(x, s2_ref[...], b2_ref[...])
    h2b = h2.astype(_BF)
    h2_ref[...] = h2b

    r = _dot_nt(h2b, wr1_ref[...]) + br1_ref[...]
    r = jnp.maximum(r, 0.0)
    sc_ref[...] = _dot_nt(r.astype(_BF), wr2_ref[...]) + br2_ref[...]


def _route_sc_kernel(nc, tokens_per_w, sc_hbm, out_hbm, in_v, out_v):
    wid = jax.lax.axis_index("s") * nc + jax.lax.axis_index("c")
    base = wid * tokens_per_w * P
    pltpu.sync_copy(sc_hbm.at[pl.ds(base, tokens_per_w * P)], in_v)
    iot = jax.lax.iota(jnp.int32, P)
    zero = jnp.zeros((P,), _F32)
    one = jnp.ones((P,), _F32)
    neg = jnp.full((P,), -1e30, _F32)

    def body(g, carry):
        idx0 = g * (P * P)
        idxs = [iot * P + (idx0 + p) for p in range(P)]
        s = [plsc.load_gather(in_v, [idxs[p]]) for p in range(P)]
        m = functools.reduce(jnp.maximum, s)
        e = [jnp.exp(sp - m) for sp in s]
        z = functools.reduce(jnp.add, e)
        cur = list(s)
        selected = [zero] * P
        for _ in range(K):
            cur_max = functools.reduce(jnp.maximum, cur)
            taken = zero
            for p in range(P):
                hit = jnp.where((cur[p] >= cur_max) & (taken < 0.5),
                                one, zero)
                selected[p] = selected[p] + hit
                taken = taken + hit
                cur[p] = jnp.where(hit > 0.5, neg, cur[p])
        w = [selected[p] * e[p] for p in range(P)]
        ssum = functools.reduce(jnp.add, w)
        inv = 1.0 / (ssum + 1e-8 * z)
        for p in range(P):
            plsc.store_scatter(out_v, [idxs[p]], w[p] * inv)
        return carry

    jax.lax.fori_loop(0, tokens_per_w // P, body, 0)
    pltpu.sync_copy(out_v, out_hbm.at[pl.ds(base, tokens_per_w * P)])


def _route_sc(scores_flat):
    info = plsc.get_sparse_core_info()
    nw = info.num_cores * info.num_subcores
    tokens_per_w = M // nw
    mesh = plsc.VectorSubcoreMesh(core_axis_name="c", subcore_axis_name="s")
    fn = pl.kernel(
        functools.partial(_route_sc_kernel, info.num_cores, tokens_per_w),
        out_type=jax.ShapeDtypeStruct((M * P,), _F32),
        mesh=mesh,
        scratch_types=[
            pltpu.VMEM((tokens_per_w * P,), _F32),
            pltpu.VMEM((tokens_per_w * P,), _F32),
        ],
        compiler_params=pltpu.CompilerParams(needs_layout_passes=False),
    )
    return fn(scores_flat)


def _epi_b_kernel(x_ref, h2_ref, pw_ref,
                  w1_ref, bfc_ref, w2_ref, bproj_ref, out_ref):
    x = x_ref[...]
    h2b = h2_ref[...]
    for i in range(P):
        hs = h2b[:, i * HPP:(i + 1) * HPP]
        w1 = w1_ref[pl.ds(i * IPP, IPP), :]
        inter = _dot_nt(hs, w1)
        inter = (inter + bfc_ref[:, i * IPP:(i + 1) * IPP]).astype(_BF)
        inter = jax.nn.gelu(inter)
        w2 = w2_ref[pl.ds(i * HPP, HPP), :]
        po = _dot_nt(inter, w2)
        po = po + bproj_ref[:, i * HPP:(i + 1) * HPP]
        out_ref[:, i * HPP:(i + 1) * HPP] = (
            x[:, i * HPP:(i + 1) * HPP] + po * pw_ref[:, i:i + 1])


@jax.jit
def _run(hidden_states, ln1_s, ln1_b, W_qkv, b_qkv, W_o, b_o, ln2_s, ln2_b,
         W_r1, b_r1, W_r2, b_r2, W_fc, b_fc, W_proj, b_proj):
    x = hidden_states.reshape(M, H)

    qk_scale = jnp.concatenate(
        [jnp.full((H, 1), 1.0 / (DH ** 0.5), _F32),
         jnp.ones((2 * H, 1), _F32)], axis=0)

    qkv = pl.pallas_call(
        _qkv_kernel,
        grid=(M // BM_QKV,),
        in_specs=[
            pl.BlockSpec((BM_QKV, H), lambda i: (i, 0)),
            pl.BlockSpec((1, H), lambda i: (0, 0)),
            pl.BlockSpec((1, H), lambda i: (0, 0)),
            pl.BlockSpec((3 * H, H), lambda i: (0, 0)),
            pl.BlockSpec((1, 3 * H), lambda i: (0, 0)),
        ],
        out_specs=pl.BlockSpec((BM_QKV, 3 * H), lambda i: (i, 0)),
        out_shape=jax.ShapeDtypeStruct((M, 3 * H), _BF),
        compiler_params=pltpu.CompilerParams(
            dimension_semantics=("parallel",)),
    )(x, ln1_s.reshape(1, H), ln1_b.reshape(1, H),
      (W_qkv * qk_scale).astype(_BF), (b_qkv * qk_scale.reshape(3 * H))
      .reshape(1, 3 * H))

    qkv3 = qkv.reshape(B, S, 3 * H)

    o = pl.pallas_call(
        _attn_kernel,
        grid=(B, NH, S // BQ),
        in_specs=[
            pl.BlockSpec((1, BQ, DH), lambda b, h, i: (b, i, h)),
            pl.BlockSpec((1, S, DH), lambda b, h, i: (b, 0, NH + h)),
            pl.BlockSpec((1, S, DH), lambda b, h, i: (b, 0, 2 * NH + h)),
        ],
        out_specs=pl.BlockSpec((1, BQ, DH), lambda b, h, i: (b, i, h)),
        out_shape=jax.ShapeDtypeStruct((B, S, H), _BF),
        scratch_shapes=[
            pltpu.VMEM((BQ, DH), _F32),
            pltpu.VMEM((BQ, 1), _F32),
            pltpu.VMEM((BQ, 1), _F32),
        ],
        compiler_params=pltpu.CompilerParams(
            dimension_semantics=("parallel", "parallel", "arbitrary")),
    )(qkv3, qkv3, qkv3)

    o2 = o.reshape(M, H)

    w1_t = jnp.concatenate(
        [W_fc[i * IPP:(i + 1) * IPP, i * HPP:(i + 1) * HPP]
         for i in range(P)], axis=0).astype(_BF)
    w2_t = jnp.concatenate(
        [W_proj[i * HPP:(i + 1) * HPP, i * IPP:(i + 1) * IPP]
         for i in range(P)], axis=0).astype(_BF)

    x2, h2b, scores = pl.pallas_call(
        _epi_a_kernel,
        grid=(M // BM_EPI,),
        in_specs=[
            pl.BlockSpec((BM_EPI, H), lambda i: (i, 0)),
            pl.BlockSpec((BM_EPI, H), lambda i: (i, 0)),
            pl.BlockSpec((H, H), lambda i: (0, 0)),
            pl.BlockSpec((1, H), lambda i: (0, 0)),
            pl.BlockSpec((1, H), lambda i: (0, 0)),
            pl.BlockSpec((1, H), lambda i: (0, 0)),
            pl.BlockSpec((RH, H), lambda i: (0, 0)),
            pl.BlockSpec((1, RH), lambda i: (0, 0)),
            pl.BlockSpec((P, RH), lambda i: (0, 0)),
            pl.BlockSpec((1, P), lambda i: (0, 0)),
        ],
        out_specs=[
            pl.BlockSpec((BM_EPI, H), lambda i: (i, 0)),
            pl.BlockSpec((BM_EPI, H), lambda i: (i, 0)),
            pl.BlockSpec((BM_EPI, P), lambda i: (i, 0)),
        ],
        out_shape=[
            jax.ShapeDtypeStruct((M, H), _F32),
            jax.ShapeDtypeStruct((M, H), _BF),
            jax.ShapeDtypeStruct((M, P), _F32),
        ],
        compiler_params=pltpu.CompilerParams(
            dimension_semantics=("parallel",)),
    )(x, o2, W_o.astype(_BF), b_o.reshape(1, H),
      ln2_s.reshape(1, H), ln2_b.reshape(1, H),
      W_r1.astype(_BF), b_r1.reshape(1, RH),
      W_r2.astype(_BF), b_r2.reshape(1, P))

    pw = _route_sc(scores.reshape(M * P)).reshape(M, P)

    out = pl.pallas_call(
        _epi_b_kernel,
        grid=(M // BM_EPI,),
        in_specs=[
            pl.BlockSpec((BM_EPI, H), lambda i: (i, 0)),
            pl.BlockSpec((BM_EPI, H), lambda i: (i, 0)),
            pl.BlockSpec((BM_EPI, P), lambda i: (i, 0)),
            pl.BlockSpec((I, HPP), lambda i: (0, 0)),
            pl.BlockSpec((1, I), lambda i: (0, 0)),
            pl.BlockSpec((H, IPP), lambda i: (0, 0)),
            pl.BlockSpec((1, H), lambda i: (0, 0)),
        ],
        out_specs=pl.BlockSpec((BM_EPI, H), lambda i: (i, 0)),
        out_shape=jax.ShapeDtypeStruct((M, H), _F32),
        compiler_params=pltpu.CompilerParams(
            dimension_semantics=("parallel",)),
    )(x2, h2b, pw, w1_t, b_fc.reshape(1, I), w2_t, b_proj.reshape(1, H))

    return out.reshape(B, S, H)


def kernel(hidden_states, ln1_s, ln1_b, W_qkv, b_qkv, W_o, b_o, ln2_s, ln2_b,
           W_r1, b_r1, W_r2, b_r2, W_fc, b_fc, W_proj, b_proj):
    return _run(hidden_states, ln1_s, ln1_b, W_qkv, b_qkv, W_o, b_o,
                ln2_s, ln2_b, W_r1, b_r1, W_r2, b_r2, W_fc, b_fc,
                W_proj, b_proj)


# speedup vs baseline: 1.2001x; 1.2001x over previous
import functools

import jax
import jax.numpy as jnp
from jax.experimental import pallas as pl
from jax.experimental.pallas import tpu as pltpu
from jax.experimental.pallas import tpu_sc as plsc

B, S, H = 2, 2048, 2048
NH = 16
DH = H // NH
P = 16
K = 4
RH = 256
I = 8192
HPP = H // P
IPP = I // P
M = B * S

BM_QKV = 512
BQ = 512
BK = 512
BM_EPI = 512

_BF = jnp.bfloat16
_F32 = jnp.float32


def _dot_nt(a, b_t):
    return jax.lax.dot_general(
        a, b_t, (((1,), (1,)), ((), ())), preferred_element_type=_F32)


def _layer_norm(x, s, b, eps=1e-5):
    m = jnp.mean(x, axis=-1, keepdims=True)
    v = jnp.mean((x - m) ** 2, axis=-1, keepdims=True)
    return (x - m) * jax.lax.rsqrt(v + eps) * s + b


def _qkv_kernel(x_ref, s_ref, b_ref, w_ref, bias_ref, o_ref):
    n = pl.program_id(0)
    h = _layer_norm(x_ref[...], s_ref[...], b_ref[...])
    acc = _dot_nt(h.astype(_BF), w_ref[...])
    o_ref[...] = (acc + bias_ref[:, pl.ds(n * H, H)]).astype(_BF)


def _attn_kernel(q_ref, k_ref, v_ref, o_ref):
    qi = pl.program_id(2)
    q = q_ref[0]

    def step(j, carry, masked):
        acc, m, l = carry
        k = k_ref[0, pl.ds(j * BK, BK), :]
        s = jax.lax.dot_general(
            q, k, (((1,), (1,)), ((), ())), preferred_element_type=_F32)
        if masked:
            qpos = jax.lax.broadcasted_iota(jnp.int32, (BQ, BK), 0)
            kpos = jax.lax.broadcasted_iota(jnp.int32, (BQ, BK), 1)
            s = jnp.where(qpos >= kpos, s, -1e30)
        m_new = jnp.maximum(m, jnp.max(s, axis=1, keepdims=True))
        p = jnp.exp(s - m_new)
        alpha = jnp.exp(m - m_new)
        l = l * alpha + jnp.sum(p, axis=1, keepdims=True)
        v = v_ref[0, pl.ds(j * BK, BK), :]
        acc = acc * alpha + jnp.dot(p.astype(_BF), v,
                                    preferred_element_type=_F32)
        return acc, m_new, l

    acc0 = jnp.zeros((BQ, DH), _F32)
    m0 = jnp.full((BQ, 1), -1e30, _F32)
    l0 = jnp.zeros((BQ, 1), _F32)
    carry = jax.lax.fori_loop(
        0, qi, lambda j, c: step(j, c, False), (acc0, m0, l0))
    acc, _, l = step(qi, carry, True)
    o_ref[0] = (acc / l).astype(_BF)


def _epi_a_kernel(res_ref, o_ref, wo_ref, bo_ref, s2_ref, b2_ref,
                  wr1_ref, br1_ref, wr2_ref, br2_ref,
                  x_ref, h2_ref, sc_ref):
    attn = _dot_nt(o_ref[...], wo_ref[...])
    x = res_ref[...] + attn + bo_ref[...]
    x_ref[...] = x
    h2 = _layer_norm(x, s2_ref[...], b2_ref[...])
    h2b = h2.astype(_BF)
    h2_ref[...] = h2b

    r = _dot_nt(h2b, wr1_ref[...]) + br1_ref[...]
    r = jnp.maximum(r, 0.0)
    sc_ref[...] = _dot_nt(r.astype(_BF), wr2_ref[...]) + br2_ref[...]


def _route_sc_kernel(nc, tokens_per_w, sc_hbm, out_hbm, in_v, out_v):
    wid = jax.lax.axis_index("s") * nc + jax.lax.axis_index("c")
    base = wid * tokens_per_w * P
    pltpu.sync_copy(sc_hbm.at[pl.ds(base, tokens_per_w * P)], in_v)
    iot = jax.lax.iota(jnp.int32, P)
    zero = jnp.zeros((P,), _F32)
    one = jnp.ones((P,), _F32)
    neg = jnp.full((P,), -1e30, _F32)

    def body(g, carry):
        idx0 = g * (P * P)
        idxs = [iot * P + (idx0 + p) for p in range(P)]
        s = [plsc.load_gather(in_v, [idxs[p]]) for p in range(P)]
        m = functools.reduce(jnp.maximum, s)
        e = [jnp.exp(sp - m) for sp in s]
        z = functools.reduce(jnp.add, e)
        cur = list(s)
        selected = [zero] * P
        for _ in range(K):
            cur_max = functools.reduce(jnp.maximum, cur)
            taken = zero
            for p in range(P):
                hit = jnp.where((cur[p] >= cur_max) & (taken < 0.5),
                                one, zero)
                selected[p] = selected[p] + hit
                taken = taken + hit
                cur[p] = jnp.where(hit > 0.5, neg, cur[p])
        w = [selected[p] * e[p] for p in range(P)]
        ssum = functools.reduce(jnp.add, w)
        inv = 1.0 / (ssum + 1e-8 * z)
        for p in range(P):
            plsc.store_scatter(out_v, [idxs[p]], w[p] * inv)
        return carry

    jax.lax.fori_loop(0, tokens_per_w // P, body, 0)
    pltpu.sync_copy(out_v, out_hbm.at[pl.ds(base, tokens_per_w * P)])


def _route_sc(scores_flat):
    info = plsc.get_sparse_core_info()
    nw = info.num_cores * info.num_subcores
    tokens_per_w = M // nw
    mesh = plsc.VectorSubcoreMesh(core_axis_name="c", subcore_axis_name="s")
    fn = pl.kernel(
        functools.partial(_route_sc_kernel, info.num_cores, tokens_per_w),
        out_type=jax.ShapeDtypeStruct((M * P,), _F32),
        mesh=mesh,
        scratch_types=[
            pltpu.VMEM((tokens_per_w * P,), _F32),
            pltpu.VMEM((tokens_per_w * P,), _F32),
        ],
        compiler_params=pltpu.CompilerParams(needs_layout_passes=False),
    )
    return fn(scores_flat)


def _epi_b_kernel(x_ref, h2_ref, pw_ref,
                  w1_ref, bfc_ref, w2_ref, bproj_ref, out_ref):
    x = x_ref[...]
    h2b = h2_ref[...]
    for i in range(P):
        hs = h2b[:, i * HPP:(i + 1) * HPP]
        w1 = w1_ref[pl.ds(i * IPP, IPP), :]
        inter = _dot_nt(hs, w1)
        inter = (inter + bfc_ref[:, i * IPP:(i + 1) * IPP]).astype(_BF)
        inter = jax.nn.gelu(inter)
        w2 = w2_ref[pl.ds(i * HPP, HPP), :]
        po = _dot_nt(inter, w2)
        po = po + bproj_ref[:, i * HPP:(i + 1) * HPP]
        out_ref[:, i * HPP:(i + 1) * HPP] = (
            x[:, i * HPP:(i + 1) * HPP] + po * pw_ref[:, i:i + 1])


@jax.jit
def _run(hidden_states, ln1_s, ln1_b, W_qkv, b_qkv, W_o, b_o, ln2_s, ln2_b,
         W_r1, b_r1, W_r2, b_r2, W_fc, b_fc, W_proj, b_proj):
    x = hidden_states.reshape(M, H)

    qk_scale = jnp.concatenate(
        [jnp.full((H, 1), 1.0 / (DH ** 0.5), _F32),
         jnp.ones((2 * H, 1), _F32)], axis=0)

    qkv = pl.pallas_call(
        _qkv_kernel,
        grid=(3, M // BM_QKV),
        in_specs=[
            pl.BlockSpec((BM_QKV, H), lambda n, i: (i, 0)),
            pl.BlockSpec((1, H), lambda n, i: (0, 0)),
            pl.BlockSpec((1, H), lambda n, i: (0, 0)),
            pl.BlockSpec((H, H), lambda n, i: (n, 0)),
            pl.BlockSpec((1, 3 * H), lambda n, i: (0, 0)),
        ],
        out_specs=pl.BlockSpec((BM_QKV, H), lambda n, i: (i, n)),
        out_shape=jax.ShapeDtypeStruct((M, 3 * H), _BF),
        compiler_params=pltpu.CompilerParams(
            dimension_semantics=("parallel", "parallel")),
    )(x, ln1_s.reshape(1, H), ln1_b.reshape(1, H),
      (W_qkv * qk_scale).astype(_BF), (b_qkv * qk_scale.reshape(3 * H))
      .reshape(1, 3 * H))

    qkv3 = qkv.reshape(B, S, 3 * H)

    o = pl.pallas_call(
        _attn_kernel,
        grid=(B, NH, S // BQ),
        in_specs=[
            pl.BlockSpec((1, BQ, DH), lambda b, h, i: (b, i, h)),
            pl.BlockSpec((1, S, DH), lambda b, h, i: (b, 0, NH + h)),
            pl.BlockSpec((1, S, DH), lambda b, h, i: (b, 0, 2 * NH + h)),
        ],
        out_specs=pl.BlockSpec((1, BQ, DH), lambda b, h, i: (b, i, h)),
        out_shape=jax.ShapeDtypeStruct((B, S, H), _BF),
        compiler_params=pltpu.CompilerParams(
            dimension_semantics=("parallel", "parallel", "arbitrary")),
    )(qkv3, qkv3, qkv3)

    o2 = o.reshape(M, H)

    w1_t = jnp.concatenate(
        [W_fc[i * IPP:(i + 1) * IPP, i * HPP:(i + 1) * HPP]
         for i in range(P)], axis=0).astype(_BF)
    w2_t = jnp.concatenate(
        [W_proj[i * HPP:(i + 1) * HPP, i * IPP:(i + 1) * IPP]
         for i in range(P)], axis=0).astype(_BF)

    x2, h2b, scores = pl.pallas_call(
        _epi_a_kernel,
        grid=(M // BM_EPI,),
        in_specs=[
            pl.BlockSpec((BM_EPI, H), lambda i: (i, 0)),
            pl.BlockSpec((BM_EPI, H), lambda i: (i, 0)),
            pl.BlockSpec((H, H), lambda i: (0, 0)),
            pl.BlockSpec((1, H), lambda i: (0, 0)),
            pl.BlockSpec((1, H), lambda i: (0, 0)),
            pl.BlockSpec((1, H), lambda i: (0, 0)),
            pl.BlockSpec((RH, H), lambda i: (0, 0)),
            pl.BlockSpec((1, RH), lambda i: (0, 0)),
            pl.BlockSpec((P, RH), lambda i: (0, 0)),
            pl.BlockSpec((1, P), lambda i: (0, 0)),
        ],
        out_specs=[
            pl.BlockSpec((BM_EPI, H), lambda i: (i, 0)),
            pl.BlockSpec((BM_EPI, H), lambda i: (i, 0)),
            pl.BlockSpec((BM_EPI, P), lambda i: (i, 0)),
        ],
        out_shape=[
            jax.ShapeDtypeStruct((M, H), _F32),
            jax.ShapeDtypeStruct((M, H), _BF),
            jax.ShapeDtypeStruct((M, P), _F32),
        ],
        compiler_params=pltpu.CompilerParams(
            dimension_semantics=("parallel",)),
    )(x, o2, W_o.astype(_BF), b_o.reshape(1, H),
      ln2_s.reshape(1, H), ln2_b.reshape(1, H),
      W_r1.astype(_BF), b_r1.reshape(1, RH),
      W_r2.astype(_BF), b_r2.reshape(1, P))

    pw = _route_sc(scores.reshape(M * P)).reshape(M, P)

    out = pl.pallas_call(
        _epi_b_kernel,
        grid=(M // BM_EPI,),
        in_specs=[
            pl.BlockSpec((BM_EPI, H), lambda i: (i, 0)),
            pl.BlockSpec((BM_EPI, H), lambda i: (i, 0)),
            pl.BlockSpec((BM_EPI, P), lambda i: (i, 0)),
            pl.BlockSpec((I, HPP), lambda i: (0, 0)),
            pl.BlockSpec((1, I), lambda i: (0, 0)),
            pl.BlockSpec((H, IPP), lambda i: (0, 0)),
            pl.BlockSpec((1, H), lambda i: (0, 0)),
        ],
        out_specs=pl.BlockSpec((BM_EPI, H), lambda i: (i, 0)),
        out_shape=jax.ShapeDtypeStruct((M, H), _F32),
        compiler_params=pltpu.CompilerParams(
            dimension_semantics=("parallel",)),
    )(x2, h2b, pw, w1_t, b_fc.reshape(1, I), w2_t, b_proj.reshape(1, H))

    return out.reshape(B, S, H)


def kernel(hidden_states, ln1_s, ln1_b, W_qkv, b_qkv, W_o, b_o, ln2_s, ln2_b,
           W_r1, b_r1, W_r2, b_r2, W_fc, b_fc, W_proj, b_proj):
    return _run(hidden_states, ln1_s, ln1_b, W_qkv, b_qkv, W_o, b_o,
                ln2_s, ln2_b, W_r1, b_r1, W_r2, b_r2, W_fc, b_fc,
                W_proj, b_proj)


# speedup vs baseline: 1.2201x; 1.0167x over previous
import functools

import jax
import jax.numpy as jnp
from jax.experimental import pallas as pl
from jax.experimental.pallas import tpu as pltpu
from jax.experimental.pallas import tpu_sc as plsc

B, S, H = 2, 2048, 2048
NH = 16
DH = H // NH
P = 16
K = 4
RH = 256
I = 8192
HPP = H // P
IPP = I // P
M = B * S

BM_QKV = 512
BQ = 512
BK = 512
BM_EPI = 512

_BF = jnp.bfloat16
_F32 = jnp.float32


def _dot_nt(a, b_t):
    return jax.lax.dot_general(
        a, b_t, (((1,), (1,)), ((), ())), preferred_element_type=_F32)


def _layer_norm(x, s, b, eps=1e-5):
    m = jnp.mean(x, axis=-1, keepdims=True)
    v = jnp.mean((x - m) ** 2, axis=-1, keepdims=True)
    return (x - m) * jax.lax.rsqrt(v + eps) * s + b


def _qkv_kernel(x_ref, s_ref, b_ref, w_ref, bias_ref, o_ref):
    h = _layer_norm(x_ref[...], s_ref[...], b_ref[...])
    acc = _dot_nt(h.astype(_BF), w_ref[...])
    o_ref[...] = (acc + bias_ref[...]).astype(_BF)


def _attn_kernel(q_ref, k_ref, v_ref, o_ref):
    qi = pl.program_id(2)
    q = q_ref[0]

    def step(j, carry, masked):
        acc, m, l = carry
        k = k_ref[0, pl.ds(j * BK, BK), :]
        s = jax.lax.dot_general(
            q, k, (((1,), (1,)), ((), ())), preferred_element_type=_F32)
        if masked:
            qpos = jax.lax.broadcasted_iota(jnp.int32, (BQ, BK), 0)
            kpos = jax.lax.broadcasted_iota(jnp.int32, (BQ, BK), 1)
            s = jnp.where(qpos >= kpos, s, -1e30)
        m_new = jnp.maximum(m, jnp.max(s, axis=1, keepdims=True))
        p = jnp.exp((s - m_new).astype(_BF))
        alpha = jnp.exp(m - m_new)
        l = l * alpha + jnp.sum(p, axis=1, keepdims=True).astype(_F32)
        v = v_ref[0, pl.ds(j * BK, BK), :]
        acc = acc * alpha + jnp.dot(p, v, preferred_element_type=_F32)
        return acc, m_new, l

    acc0 = jnp.zeros((BQ, DH), _F32)
    m0 = jnp.full((BQ, 1), -1e30, _F32)
    l0 = jnp.zeros((BQ, 1), _F32)
    carry = jax.lax.fori_loop(
        0, qi, lambda j, c: step(j, c, False), (acc0, m0, l0))
    acc, _, l = step(qi, carry, True)
    o_ref[0] = (acc / l).astype(_BF)


def _epi_a_kernel(res_ref, o_ref, wo_ref, bo_ref, s2_ref, b2_ref,
                  wr1_ref, br1_ref, wr2_ref, br2_ref,
                  x_ref, h2_ref, sc_ref):
    attn = _dot_nt(o_ref[...], wo_ref[...])
    x = res_ref[...] + attn + bo_ref[...]
    x_ref[...] = x
    h2 = _layer_norm(x, s2_ref[...], b2_ref[...])
    h2b = h2.astype(_BF)
    h2_ref[...] = h2b

    r = _dot_nt(h2b, wr1_ref[...]) + br1_ref[...]
    r = jnp.maximum(r, 0.0)
    sc_ref[...] = _dot_nt(r.astype(_BF), wr2_ref[...]) + br2_ref[...]


def _route_sc_kernel(nc, tokens_per_w, sc_hbm, out_hbm, in_v, out_v):
    wid = jax.lax.axis_index("s") * nc + jax.lax.axis_index("c")
    base = wid * tokens_per_w * P
    pltpu.sync_copy(sc_hbm.at[pl.ds(base, tokens_per_w * P)], in_v)
    iot = jax.lax.iota(jnp.int32, P)
    zero = jnp.zeros((P,), _F32)
    one = jnp.ones((P,), _F32)
    neg = jnp.full((P,), -1e30, _F32)

    def body(g, carry):
        idx0 = g * (P * P)
        idxs = [iot * P + (idx0 + p) for p in range(P)]
        s = [plsc.load_gather(in_v, [idxs[p]]) for p in range(P)]
        m = functools.reduce(jnp.maximum, s)
        e = [jnp.exp(sp - m) for sp in s]
        z = functools.reduce(jnp.add, e)
        cur = list(s)
        selected = [zero] * P
        for _ in range(K):
            cur_max = functools.reduce(jnp.maximum, cur)
            taken = zero
            for p in range(P):
                hit = jnp.where((cur[p] >= cur_max) & (taken < 0.5),
                                one, zero)
                selected[p] = selected[p] + hit
                taken = taken + hit
                cur[p] = jnp.where(hit > 0.5, neg, cur[p])
        w = [selected[p] * e[p] for p in range(P)]
        ssum = functools.reduce(jnp.add, w)
        inv = 1.0 / (ssum + 1e-8 * z)
        for p in range(P):
            plsc.store_scatter(out_v, [idxs[p]], w[p] * inv)
        return carry

    jax.lax.fori_loop(0, tokens_per_w // P, body, 0)
    pltpu.sync_copy(out_v, out_hbm.at[pl.ds(base, tokens_per_w * P)])


def _route_sc(scores_flat):
    info = plsc.get_sparse_core_info()
    nw = info.num_cores * info.num_subcores
    tokens_per_w = M // nw
    mesh = plsc.VectorSubcoreMesh(core_axis_name="c", subcore_axis_name="s")
    fn = pl.kernel(
        functools.partial(_route_sc_kernel, info.num_cores, tokens_per_w),
        out_type=jax.ShapeDtypeStruct((M * P,), _F32),
        mesh=mesh,
        scratch_types=[
            pltpu.VMEM((tokens_per_w * P,), _F32),
            pltpu.VMEM((tokens_per_w * P,), _F32),
        ],
        compiler_params=pltpu.CompilerParams(needs_layout_passes=False),
    )
    return fn(scores_flat)


def _epi_b_kernel(x_ref, h2_ref, pw_ref,
                  w1_ref, bfc_ref, w2_ref, bproj_ref, out_ref):
    x = x_ref[...]
    h2b = h2_ref[...]
    for i in range(P):
        hs = h2b[:, i * HPP:(i + 1) * HPP]
        w1 = w1_ref[pl.ds(i * IPP, IPP), :]
        inter = _dot_nt(hs, w1)
        inter = (inter + bfc_ref[:, i * IPP:(i + 1) * IPP]).astype(_BF)
        inter = jax.nn.gelu(inter)
        w2 = w2_ref[pl.ds(i * HPP, HPP), :]
        po = _dot_nt(inter, w2)
        po = po + bproj_ref[:, i * HPP:(i + 1) * HPP]
        out_ref[:, i * HPP:(i + 1) * HPP] = (
            x[:, i * HPP:(i + 1) * HPP] + po * pw_ref[:, i:i + 1])


@jax.jit
def _run(hidden_states, ln1_s, ln1_b, W_qkv, b_qkv, W_o, b_o, ln2_s, ln2_b,
         W_r1, b_r1, W_r2, b_r2, W_fc, b_fc, W_proj, b_proj):
    x = hidden_states.reshape(M, H)

    qk_scale = jnp.concatenate(
        [jnp.full((H, 1), 1.0 / (DH ** 0.5), _F32),
         jnp.ones((2 * H, 1), _F32)], axis=0)

    qkv = pl.pallas_call(
        _qkv_kernel,
        grid=(M // BM_QKV,),
        in_specs=[
            pl.BlockSpec((BM_QKV, H), lambda i: (i, 0)),
            pl.BlockSpec((1, H), lambda i: (0, 0)),
            pl.BlockSpec((1, H), lambda i: (0, 0)),
            pl.BlockSpec((3 * H, H), lambda i: (0, 0)),
            pl.BlockSpec((1, 3 * H), lambda i: (0, 0)),
        ],
        out_specs=pl.BlockSpec((BM_QKV, 3 * H), lambda i: (i, 0)),
        out_shape=jax.ShapeDtypeStruct((M, 3 * H), _BF),
        compiler_params=pltpu.CompilerParams(
            dimension_semantics=("parallel",)),
    )(x, ln1_s.reshape(1, H), ln1_b.reshape(1, H),
      (W_qkv * qk_scale).astype(_BF), (b_qkv * qk_scale.reshape(3 * H))
      .reshape(1, 3 * H))

    qkv3 = qkv.reshape(B, S, 3 * H)

    o = pl.pallas_call(
        _attn_kernel,
        grid=(B, NH, S // BQ),
        in_specs=[
            pl.BlockSpec((1, BQ, DH), lambda b, h, i: (b, i, h)),
            pl.BlockSpec((1, S, DH), lambda b, h, i: (b, 0, NH + h)),
            pl.BlockSpec((1, S, DH), lambda b, h, i: (b, 0, 2 * NH + h)),
        ],
        out_specs=pl.BlockSpec((1, BQ, DH), lambda b, h, i: (b, i, h)),
        out_shape=jax.ShapeDtypeStruct((B, S, H), _BF),
        compiler_params=pltpu.CompilerParams(
            dimension_semantics=("parallel", "parallel", "arbitrary")),
    )(qkv3, qkv3, qkv3)

    o2 = o.reshape(M, H)

    w1_t = jnp.concatenate(
        [W_fc[i * IPP:(i + 1) * IPP, i * HPP:(i + 1) * HPP]
         for i in range(P)], axis=0).astype(_BF)
    w2_t = jnp.concatenate(
        [W_proj[i * HPP:(i + 1) * HPP, i * IPP:(i + 1) * IPP]
         for i in range(P)], axis=0).astype(_BF)

    x2, h2b, scores = pl.pallas_call(
        _epi_a_kernel,
        grid=(M // BM_EPI,),
        in_specs=[
            pl.BlockSpec((BM_EPI, H), lambda i: (i, 0)),
            pl.BlockSpec((BM_EPI, H), lambda i: (i, 0)),
            pl.BlockSpec((H, H), lambda i: (0, 0)),
            pl.BlockSpec((1, H), lambda i: (0, 0)),
            pl.BlockSpec((1, H), lambda i: (0, 0)),
            pl.BlockSpec((1, H), lambda i: (0, 0)),
            pl.BlockSpec((RH, H), lambda i: (0, 0)),
            pl.BlockSpec((1, RH), lambda i: (0, 0)),
            pl.BlockSpec((P, RH), lambda i: (0, 0)),
            pl.BlockSpec((1, P), lambda i: (0, 0)),
        ],
        out_specs=[
            pl.BlockSpec((BM_EPI, H), lambda i: (i, 0)),
            pl.BlockSpec((BM_EPI, H), lambda i: (i, 0)),
            pl.BlockSpec((BM_EPI, P), lambda i: (i, 0)),
        ],
        out_shape=[
            jax.ShapeDtypeStruct((M, H), _F32),
            jax.ShapeDtypeStruct((M, H), _BF),
            jax.ShapeDtypeStruct((M, P), _F32),
        ],
        compiler_params=pltpu.CompilerParams(
            dimension_semantics=("parallel",)),
    )(x, o2, W_o.astype(_BF), b_o.reshape(1, H),
      ln2_s.reshape(1, H), ln2_b.reshape(1, H),
      W_r1.astype(_BF), b_r1.reshape(1, RH),
      W_r2.astype(_BF), b_r2.reshape(1, P))

    pw = _route_sc(scores.reshape(M * P)).reshape(M, P)

    out = pl.pallas_call(
        _epi_b_kernel,
        grid=(M // BM_EPI,),
        in_specs=[
            pl.BlockSpec((BM_EPI, H), lambda i: (i, 0)),
            pl.BlockSpec((BM_EPI, H), lambda i: (i, 0)),
            pl.BlockSpec((BM_EPI, P), lambda i: (i, 0)),
            pl.BlockSpec((I, HPP), lambda i: (0, 0)),
            pl.BlockSpec((1, I), lambda i: (0, 0)),
            pl.BlockSpec((H, IPP), lambda i: (0, 0)),
            pl.BlockSpec((1, H), lambda i: (0, 0)),
        ],
        out_specs=pl.BlockSpec((BM_EPI, H), lambda i: (i, 0)),
        out_shape=jax.ShapeDtypeStruct((M, H), _F32),
        compiler_params=pltpu.CompilerParams(
            dimension_semantics=("parallel",)),
    )(x2, h2b, pw, w1_t, b_fc.reshape(1, I), w2_t, b_proj.reshape(1, H))

    return out.reshape(B, S, H)


def kernel(hidden_states, ln1_s, ln1_b, W_qkv, b_qkv, W_o, b_o, ln2_s, ln2_b,
           W_r1, b_r1, W_r2, b_r2, W_fc, b_fc, W_proj, b_proj):
    return _run(hidden_states, ln1_s, ln1_b, W_qkv, b_qkv, W_o, b_o,
                ln2_s, ln2_b, W_r1, b_r1, W_r2, b_r2, W_fc, b_fc,
                W_proj, b_proj)


# speedup vs baseline: 1.2354x; 1.0125x over previous
import functools

import jax
import jax.numpy as jnp
from jax.experimental import pallas as pl
from jax.experimental.pallas import tpu as pltpu
from jax.experimental.pallas import tpu_sc as plsc

B, S, H = 2, 2048, 2048
NH = 16
DH = H // NH
P = 16
K = 4
RH = 256
I = 8192
HPP = H // P
IPP = I // P
M = B * S

BM_QKV = 512
BQ = 512
BK = 512
BM_EPI = 512

_BF = jnp.bfloat16
_F32 = jnp.float32


def _dot_nt(a, b_t):
    return jax.lax.dot_general(
        a, b_t, (((1,), (1,)), ((), ())), preferred_element_type=_F32)


def _layer_norm(x, s, b, eps=1e-5):
    m = jnp.mean(x, axis=-1, keepdims=True)
    v = jnp.mean((x - m) ** 2, axis=-1, keepdims=True)
    return (x - m) * jax.lax.rsqrt(v + eps) * s + b


def _qkv_kernel(x_ref, s_ref, b_ref, w_ref, bias_ref, o_ref):
    h = _layer_norm(x_ref[...], s_ref[...], b_ref[...])
    acc = _dot_nt(h.astype(_BF), w_ref[...])
    o_ref[...] = (acc + bias_ref[...]).astype(_BF)


def _attn_kernel(q_ref, k_ref, v_ref, o_ref):
    qi = pl.program_id(2)
    q = q_ref[0]

    def step(j, carry, masked):
        acc, m, l = carry
        k = k_ref[0, pl.ds(j * BK, BK), :]
        s = jax.lax.dot_general(
            q, k, (((1,), (1,)), ((), ())), preferred_element_type=_F32)
        if masked:
            qpos = jax.lax.broadcasted_iota(jnp.int32, (BQ, BK), 0)
            kpos = jax.lax.broadcasted_iota(jnp.int32, (BQ, BK), 1)
            s = jnp.where(qpos >= kpos, s, -1e30)
        m_new = jnp.maximum(m, jnp.max(s, axis=1, keepdims=True))
        p = jnp.exp((s - m_new).astype(_BF))
        alpha = jnp.exp(m - m_new)
        l = l * alpha + jnp.sum(p, axis=1, keepdims=True).astype(_F32)
        v = v_ref[0, pl.ds(j * BK, BK), :]
        acc = acc * alpha + jnp.dot(p, v, preferred_element_type=_F32)
        return acc, m_new, l

    acc0 = jnp.zeros((BQ, DH), _F32)
    m0 = jnp.full((BQ, 1), -1e30, _F32)
    l0 = jnp.zeros((BQ, 1), _F32)
    def pair(jj, c):
        return step(2 * jj + 1, step(2 * jj, c, False), False)

    carry = jax.lax.fori_loop(0, qi // 2, pair, (acc0, m0, l0))
    carry = jax.lax.cond(
        qi % 2 == 1, lambda c: step(qi - 1, c, False), lambda c: c, carry)
    acc, _, l = step(qi, carry, True)
    o_ref[0] = (acc / l).astype(_BF)


def _epi_a_kernel(res_ref, o_ref, wo_ref, bo_ref, s2_ref, b2_ref,
                  wr1_ref, br1_ref, wr2_ref, br2_ref,
                  x_ref, h2_ref, sc_ref):
    attn = _dot_nt(o_ref[...], wo_ref[...])
    x = res_ref[...] + attn + bo_ref[...]
    x_ref[...] = x
    h2 = _layer_norm(x, s2_ref[...], b2_ref[...])
    h2b = h2.astype(_BF)
    h2_ref[...] = h2b

    r = _dot_nt(h2b, wr1_ref[...]) + br1_ref[...]
    r = jnp.maximum(r, 0.0)
    sc_ref[...] = _dot_nt(r.astype(_BF), wr2_ref[...]) + br2_ref[...]


def _route_sc_kernel(nc, tokens_per_w, sc_hbm, out_hbm, in_v, out_v):
    wid = jax.lax.axis_index("s") * nc + jax.lax.axis_index("c")
    base = wid * tokens_per_w * P
    pltpu.sync_copy(sc_hbm.at[pl.ds(base, tokens_per_w * P)], in_v)
    iot = jax.lax.iota(jnp.int32, P)
    zero = jnp.zeros((P,), _F32)
    one = jnp.ones((P,), _F32)
    neg = jnp.full((P,), -1e30, _F32)

    def body(g, carry):
        idx0 = g * (P * P)
        idxs = [iot * P + (idx0 + p) for p in range(P)]
        s = [plsc.load_gather(in_v, [idxs[p]]) for p in range(P)]
        m = functools.reduce(jnp.maximum, s)
        e = [jnp.exp(sp - m) for sp in s]
        z = functools.reduce(jnp.add, e)
        cur = list(s)
        selected = [zero] * P
        for _ in range(K):
            cur_max = functools.reduce(jnp.maximum, cur)
            taken = zero
            for p in range(P):
                hit = jnp.where((cur[p] >= cur_max) & (taken < 0.5),
                                one, zero)
                selected[p] = selected[p] + hit
                taken = taken + hit
                cur[p] = jnp.where(hit > 0.5, neg, cur[p])
        w = [selected[p] * e[p] for p in range(P)]
        ssum = functools.reduce(jnp.add, w)
        inv = 1.0 / (ssum + 1e-8 * z)
        for p in range(P):
            plsc.store_scatter(out_v, [idxs[p]], w[p] * inv)
        return carry

    jax.lax.fori_loop(0, tokens_per_w // P, body, 0)
    pltpu.sync_copy(out_v, out_hbm.at[pl.ds(base, tokens_per_w * P)])


def _route_sc(scores_flat):
    info = plsc.get_sparse_core_info()
    nw = info.num_cores * info.num_subcores
    tokens_per_w = M // nw
    mesh = plsc.VectorSubcoreMesh(core_axis_name="c", subcore_axis_name="s")
    fn = pl.kernel(
        functools.partial(_route_sc_kernel, info.num_cores, tokens_per_w),
        out_type=jax.ShapeDtypeStruct((M * P,), _F32),
        mesh=mesh,
        scratch_types=[
            pltpu.VMEM((tokens_per_w * P,), _F32),
            pltpu.VMEM((tokens_per_w * P,), _F32),
        ],
        compiler_params=pltpu.CompilerParams(needs_layout_passes=False),
    )
    return fn(scores_flat)


def _epi_b_kernel(x_ref, h2_ref, pw_ref,
                  w1_ref, bfc_ref, w2_ref, bproj_ref, out_ref):
    x = x_ref[...]
    h2b = h2_ref[...]
    for i in range(P):
        hs = h2b[:, i * HPP:(i + 1) * HPP]
        w1 = w1_ref[pl.ds(i * IPP, IPP), :]
        inter = _dot_nt(hs, w1)
        inter = (inter + bfc_ref[:, i * IPP:(i + 1) * IPP]).astype(_BF)
        inter = jax.nn.gelu(inter)
        w2 = w2_ref[pl.ds(i * HPP, HPP), :]
        po = _dot_nt(inter, w2)
        po = po + bproj_ref[:, i * HPP:(i + 1) * HPP]
        out_ref[:, i * HPP:(i + 1) * HPP] = (
            x[:, i * HPP:(i + 1) * HPP] + po * pw_ref[:, i:i + 1])


@jax.jit
def _run(hidden_states, ln1_s, ln1_b, W_qkv, b_qkv, W_o, b_o, ln2_s, ln2_b,
         W_r1, b_r1, W_r2, b_r2, W_fc, b_fc, W_proj, b_proj):
    x = hidden_states.reshape(M, H)

    qk_scale = jnp.concatenate(
        [jnp.full((H, 1), 1.0 / (DH ** 0.5), _F32),
         jnp.ones((2 * H, 1), _F32)], axis=0)

    qkv = pl.pallas_call(
        _qkv_kernel,
        grid=(M // BM_QKV,),
        in_specs=[
            pl.BlockSpec((BM_QKV, H), lambda i: (i, 0)),
            pl.BlockSpec((1, H), lambda i: (0, 0)),
            pl.BlockSpec((1, H), lambda i: (0, 0)),
            pl.BlockSpec((3 * H, H), lambda i: (0, 0)),
            pl.BlockSpec((1, 3 * H), lambda i: (0, 0)),
        ],
        out_specs=pl.BlockSpec((BM_QKV, 3 * H), lambda i: (i, 0)),
        out_shape=jax.ShapeDtypeStruct((M, 3 * H), _BF),
        compiler_params=pltpu.CompilerParams(
            dimension_semantics=("parallel",)),
    )(x, ln1_s.reshape(1, H), ln1_b.reshape(1, H),
      (W_qkv * qk_scale).astype(_BF), (b_qkv * qk_scale.reshape(3 * H))
      .reshape(1, 3 * H))

    qkv3 = qkv.reshape(B, S, 3 * H)

    o = pl.pallas_call(
        _attn_kernel,
        grid=(B, NH, S // BQ),
        in_specs=[
            pl.BlockSpec((1, BQ, DH), lambda b, h, i: (b, i, h)),
            pl.BlockSpec((1, S, DH), lambda b, h, i: (b, 0, NH + h)),
            pl.BlockSpec((1, S, DH), lambda b, h, i: (b, 0, 2 * NH + h)),
        ],
        out_specs=pl.BlockSpec((1, BQ, DH), lambda b, h, i: (b, i, h)),
        out_shape=jax.ShapeDtypeStruct((B, S, H), _BF),
        compiler_params=pltpu.CompilerParams(
            dimension_semantics=("parallel", "parallel", "arbitrary")),
    )(qkv3, qkv3, qkv3)

    o2 = o.reshape(M, H)

    w1_t = jnp.concatenate(
        [W_fc[i * IPP:(i + 1) * IPP, i * HPP:(i + 1) * HPP]
         for i in range(P)], axis=0).astype(_BF)
    w2_t = jnp.concatenate(
        [W_proj[i * HPP:(i + 1) * HPP, i * IPP:(i + 1) * IPP]
         for i in range(P)], axis=0).astype(_BF)

    x2, h2b, scores = pl.pallas_call(
        _epi_a_kernel,
        grid=(M // BM_EPI,),
        in_specs=[
            pl.BlockSpec((BM_EPI, H), lambda i: (i, 0)),
            pl.BlockSpec((BM_EPI, H), lambda i: (i, 0)),
            pl.BlockSpec((H, H), lambda i: (0, 0)),
            pl.BlockSpec((1, H), lambda i: (0, 0)),
            pl.BlockSpec((1, H), lambda i: (0, 0)),
            pl.BlockSpec((1, H), lambda i: (0, 0)),
            pl.BlockSpec((RH, H), lambda i: (0, 0)),
            pl.BlockSpec((1, RH), lambda i: (0, 0)),
            pl.BlockSpec((P, RH), lambda i: (0, 0)),
            pl.BlockSpec((1, P), lambda i: (0, 0)),
        ],
        out_specs=[
            pl.BlockSpec((BM_EPI, H), lambda i: (i, 0)),
            pl.BlockSpec((BM_EPI, H), lambda i: (i, 0)),
            pl.BlockSpec((BM_EPI, P), lambda i: (i, 0)),
        ],
        out_shape=[
            jax.ShapeDtypeStruct((M, H), _F32),
            jax.ShapeDtypeStruct((M, H), _BF),
            jax.ShapeDtypeStruct((M, P), _F32),
        ],
        compiler_params=pltpu.CompilerParams(
            dimension_semantics=("parallel",)),
    )(x, o2, W_o.astype(_BF), b_o.reshape(1, H),
      ln2_s.reshape(1, H), ln2_b.reshape(1, H),
      W_r1.astype(_BF), b_r1.reshape(1, RH),
      W_r2.astype(_BF), b_r2.reshape(1, P))

    pw = _route_sc(scores.reshape(M * P)).reshape(M, P)

    out = pl.pallas_call(
        _epi_b_kernel,
        grid=(M // BM_EPI,),
        in_specs=[
            pl.BlockSpec((BM_EPI, H), lambda i: (i, 0)),
            pl.BlockSpec((BM_EPI, H), lambda i: (i, 0)),
            pl.BlockSpec((BM_EPI, P), lambda i: (i, 0)),
            pl.BlockSpec((I, HPP), lambda i: (0, 0)),
            pl.BlockSpec((1, I), lambda i: (0, 0)),
            pl.BlockSpec((H, IPP), lambda i: (0, 0)),
            pl.BlockSpec((1, H), lambda i: (0, 0)),
        ],
        out_specs=pl.BlockSpec((BM_EPI, H), lambda i: (i, 0)),
        out_shape=jax.ShapeDtypeStruct((M, H), _F32),
        compiler_params=pltpu.CompilerParams(
            dimension_semantics=("parallel",)),
    )(x2, h2b, pw, w1_t, b_fc.reshape(1, I), w2_t, b_proj.reshape(1, H))

    return out.reshape(B, S, H)


def kernel(hidden_states, ln1_s, ln1_b, W_qkv, b_qkv, W_o, b_o, ln2_s, ln2_b,
           W_r1, b_r1, W_r2, b_r2, W_fc, b_fc, W_proj, b_proj):
    return _run(hidden_states, ln1_s, ln1_b, W_qkv, b_qkv, W_o, b_o,
                ln2_s, ln2_b, W_r1, b_r1, W_r2, b_r2, W_fc, b_fc,
                W_proj, b_proj)


# speedup vs baseline: 1.3423x; 1.0865x over previous
import functools

import jax
import jax.numpy as jnp
from jax.experimental import pallas as pl
from jax.experimental.pallas import tpu as pltpu
from jax.experimental.pallas import tpu_sc as plsc

B, S, H = 2, 2048, 2048
NH = 16
DH = H // NH
P = 16
K = 4
RH = 256
I = 8192
HPP = H // P
IPP = I // P
M = B * S

BM_QKV = 512
BQ = 1024
BK = 512
BM_EPI = 512

_BF = jnp.bfloat16
_F32 = jnp.float32


def _dot_nt(a, b_t):
    return jax.lax.dot_general(
        a, b_t, (((1,), (1,)), ((), ())), preferred_element_type=_F32)


def _layer_norm(x, s, b, eps=1e-5):
    m = jnp.mean(x, axis=-1, keepdims=True)
    v = jnp.mean((x - m) ** 2, axis=-1, keepdims=True)
    return (x - m) * jax.lax.rsqrt(v + eps) * s + b


def _qkv_kernel(x_ref, s_ref, b_ref, w_ref, bias_ref, o_ref):
    h = _layer_norm(x_ref[...], s_ref[...], b_ref[...])
    acc = _dot_nt(h.astype(_BF), w_ref[...])
    o_ref[...] = (acc + bias_ref[...]).astype(_BF)


def _attn_kernel(q_ref, k_ref, v_ref, o_ref):
    qi = pl.program_id(2)
    q = q_ref[0]
    rq = BQ // BK

    def step(j, carry, masked):
        acc, m, l = carry
        k = k_ref[0, pl.ds(j * BK, BK), :]
        s = jax.lax.dot_general(
            q, k, (((1,), (1,)), ((), ())), preferred_element_type=_F32)
        if masked:
            qpos = qi * BQ + jax.lax.broadcasted_iota(
                jnp.int32, (BQ, BK), 0)
            kpos = j * BK + jax.lax.broadcasted_iota(
                jnp.int32, (BQ, BK), 1)
            s = jnp.where(qpos >= kpos, s, -1e30)
        m_new = jnp.maximum(m, jnp.max(s, axis=1, keepdims=True))
        p = jnp.exp((s - m_new).astype(_BF))
        alpha = jnp.exp(m - m_new)
        l = l * alpha + jnp.sum(p, axis=1, keepdims=True).astype(_F32)
        v = v_ref[0, pl.ds(j * BK, BK), :]
        acc = acc * alpha + jnp.dot(p, v, preferred_element_type=_F32)
        return acc, m_new, l

    acc0 = jnp.zeros((BQ, DH), _F32)
    m0 = jnp.full((BQ, 1), -1e30, _F32)
    l0 = jnp.zeros((BQ, 1), _F32)
    def pair(jj, c):
        return step(2 * jj + 1, step(2 * jj, c, False), False)

    carry = jax.lax.fori_loop(0, qi * rq // 2, pair, (acc0, m0, l0))
    for jj in range(rq):
        carry = step(qi * rq + jj, carry, True)
    acc, _, l = carry
    o_ref[0] = (acc / l).astype(_BF)


def _epi_a_kernel(res_ref, o_ref, wo_ref, bo_ref, s2_ref, b2_ref,
                  wr1_ref, br1_ref, wr2_ref, br2_ref,
                  x_ref, h2_ref, sc_ref):
    attn = _dot_nt(o_ref[...], wo_ref[...])
    x = res_ref[...] + attn + bo_ref[...]
    x_ref[...] = x
    h2 = _layer_norm(x, s2_ref[...], b2_ref[...])
    h2b = h2.astype(_BF)
    h2_ref[...] = h2b

    r = _dot_nt(h2b, wr1_ref[...]) + br1_ref[...]
    r = jnp.maximum(r, 0.0)
    sc_ref[...] = _dot_nt(r.astype(_BF), wr2_ref[...]) + br2_ref[...]


def _route_sc_kernel(nc, tokens_per_w, sc_hbm, out_hbm, in_v, out_v):
    wid = jax.lax.axis_index("s") * nc + jax.lax.axis_index("c")
    base = wid * tokens_per_w * P
    pltpu.sync_copy(sc_hbm.at[pl.ds(base, tokens_per_w * P)], in_v)
    iot = jax.lax.iota(jnp.int32, P)
    zero = jnp.zeros((P,), _F32)
    one = jnp.ones((P,), _F32)
    neg = jnp.full((P,), -1e30, _F32)

    def body(g, carry):
        idx0 = g * (P * P)
        idxs = [iot * P + (idx0 + p) for p in range(P)]
        s = [plsc.load_gather(in_v, [idxs[p]]) for p in range(P)]
        m = functools.reduce(jnp.maximum, s)
        e = [jnp.exp(sp - m) for sp in s]
        z = functools.reduce(jnp.add, e)
        cur = list(s)
        selected = [zero] * P
        for _ in range(K):
            cur_max = functools.reduce(jnp.maximum, cur)
            taken = zero
            for p in range(P):
                hit = jnp.where((cur[p] >= cur_max) & (taken < 0.5),
                                one, zero)
                selected[p] = selected[p] + hit
                taken = taken + hit
                cur[p] = jnp.where(hit > 0.5, neg, cur[p])
        w = [selected[p] * e[p] for p in range(P)]
        ssum = functools.reduce(jnp.add, w)
        inv = 1.0 / (ssum + 1e-8 * z)
        for p in range(P):
            plsc.store_scatter(out_v, [idxs[p]], w[p] * inv)
        return carry

    jax.lax.fori_loop(0, tokens_per_w // P, body, 0)
    pltpu.sync_copy(out_v, out_hbm.at[pl.ds(base, tokens_per_w * P)])


def _route_sc(scores_flat):
    info = plsc.get_sparse_core_info()
    nw = info.num_cores * info.num_subcores
    tokens_per_w = M // nw
    mesh = plsc.VectorSubcoreMesh(core_axis_name="c", subcore_axis_name="s")
    fn = pl.kernel(
        functools.partial(_route_sc_kernel, info.num_cores, tokens_per_w),
        out_type=jax.ShapeDtypeStruct((M * P,), _F32),
        mesh=mesh,
        scratch_types=[
            pltpu.VMEM((tokens_per_w * P,), _F32),
            pltpu.VMEM((tokens_per_w * P,), _F32),
        ],
        compiler_params=pltpu.CompilerParams(needs_layout_passes=False),
    )
    return fn(scores_flat)


def _epi_b_kernel(x_ref, h2_ref, pw_ref,
                  w1_ref, bfc_ref, w2_ref, bproj_ref, out_ref):
    x = x_ref[...]
    h2b = h2_ref[...]
    for i in range(P):
        hs = h2b[:, i * HPP:(i + 1) * HPP]
        w1 = w1_ref[pl.ds(i * IPP, IPP), :]
        inter = _dot_nt(hs, w1)
        inter = (inter + bfc_ref[:, i * IPP:(i + 1) * IPP]).astype(_BF)
        inter = jax.nn.gelu(inter)
        w2 = w2_ref[pl.ds(i * HPP, HPP), :]
        po = _dot_nt(inter, w2)
        po = po + bproj_ref[:, i * HPP:(i + 1) * HPP]
        out_ref[:, i * HPP:(i + 1) * HPP] = (
            x[:, i * HPP:(i + 1) * HPP] + po * pw_ref[:, i:i + 1])


@jax.jit
def _run(hidden_states, ln1_s, ln1_b, W_qkv, b_qkv, W_o, b_o, ln2_s, ln2_b,
         W_r1, b_r1, W_r2, b_r2, W_fc, b_fc, W_proj, b_proj):
    x = hidden_states.reshape(M, H)

    qk_scale = jnp.concatenate(
        [jnp.full((H, 1), 1.0 / (DH ** 0.5), _F32),
         jnp.ones((2 * H, 1), _F32)], axis=0)

    qkv = pl.pallas_call(
        _qkv_kernel,
        grid=(M // BM_QKV,),
        in_specs=[
            pl.BlockSpec((BM_QKV, H), lambda i: (i, 0)),
            pl.BlockSpec((1, H), lambda i: (0, 0)),
            pl.BlockSpec((1, H), lambda i: (0, 0)),
            pl.BlockSpec((3 * H, H), lambda i: (0, 0)),
            pl.BlockSpec((1, 3 * H), lambda i: (0, 0)),
        ],
        out_specs=pl.BlockSpec((BM_QKV, 3 * H), lambda i: (i, 0)),
        out_shape=jax.ShapeDtypeStruct((M, 3 * H), _BF),
        compiler_params=pltpu.CompilerParams(
            dimension_semantics=("parallel",)),
    )(x, ln1_s.reshape(1, H), ln1_b.reshape(1, H),
      (W_qkv * qk_scale).astype(_BF), (b_qkv * qk_scale.reshape(3 * H))
      .reshape(1, 3 * H))

    qkv3 = qkv.reshape(B, S, 3 * H)

    o = pl.pallas_call(
        _attn_kernel,
        grid=(B, NH, S // BQ),
        in_specs=[
            pl.BlockSpec((1, BQ, DH), lambda b, h, i: (b, i, h)),
            pl.BlockSpec((1, S, DH), lambda b, h, i: (b, 0, NH + h)),
            pl.BlockSpec((1, S, DH), lambda b, h, i: (b, 0, 2 * NH + h)),
        ],
        out_specs=pl.BlockSpec((1, BQ, DH), lambda b, h, i: (b, i, h)),
        out_shape=jax.ShapeDtypeStruct((B, S, H), _BF),
        compiler_params=pltpu.CompilerParams(
            dimension_semantics=("parallel", "parallel", "arbitrary")),
    )(qkv3, qkv3, qkv3)

    o2 = o.reshape(M, H)

    w1_t = jnp.concatenate(
        [W_fc[i * IPP:(i + 1) * IPP, i * HPP:(i + 1) * HPP]
         for i in range(P)], axis=0).astype(_BF)
    w2_t = jnp.concatenate(
        [W_proj[i * HPP:(i + 1) * HPP, i * IPP:(i + 1) * IPP]
         for i in range(P)], axis=0).astype(_BF)

    x2, h2b, scores = pl.pallas_call(
        _epi_a_kernel,
        grid=(M // BM_EPI,),
        in_specs=[
            pl.BlockSpec((BM_EPI, H), lambda i: (i, 0)),
            pl.BlockSpec((BM_EPI, H), lambda i: (i, 0)),
            pl.BlockSpec((H, H), lambda i: (0, 0)),
            pl.BlockSpec((1, H), lambda i: (0, 0)),
            pl.BlockSpec((1, H), lambda i: (0, 0)),
            pl.BlockSpec((1, H), lambda i: (0, 0)),
            pl.BlockSpec((RH, H), lambda i: (0, 0)),
            pl.BlockSpec((1, RH), lambda i: (0, 0)),
            pl.BlockSpec((P, RH), lambda i: (0, 0)),
            pl.BlockSpec((1, P), lambda i: (0, 0)),
        ],
        out_specs=[
            pl.BlockSpec((BM_EPI, H), lambda i: (i, 0)),
            pl.BlockSpec((BM_EPI, H), lambda i: (i, 0)),
            pl.BlockSpec((BM_EPI, P), lambda i: (i, 0)),
        ],
        out_shape=[
            jax.ShapeDtypeStruct((M, H), _F32),
            jax.ShapeDtypeStruct((M, H), _BF),
            jax.ShapeDtypeStruct((M, P), _F32),
        ],
        compiler_params=pltpu.CompilerParams(
            dimension_semantics=("parallel",)),
    )(x, o2, W_o.astype(_BF), b_o.reshape(1, H),
      ln2_s.reshape(1, H), ln2_b.reshape(1, H),
      W_r1.astype(_BF), b_r1.reshape(1, RH),
      W_r2.astype(_BF), b_r2.reshape(1, P))

    pw = _route_sc(scores.reshape(M * P)).reshape(M, P)

    out = pl.pallas_call(
        _epi_b_kernel,
        grid=(M // BM_EPI,),
        in_specs=[
            pl.BlockSpec((BM_EPI, H), lambda i: (i, 0)),
            pl.BlockSpec((BM_EPI, H), lambda i: (i, 0)),
            pl.BlockSpec((BM_EPI, P), lambda i: (i, 0)),
            pl.BlockSpec((I, HPP), lambda i: (0, 0)),
            pl.BlockSpec((1, I), lambda i: (0, 0)),
            pl.BlockSpec((H, IPP), lambda i: (0, 0)),
            pl.BlockSpec((1, H), lambda i: (0, 0)),
        ],
        out_specs=pl.BlockSpec((BM_EPI, H), lambda i: (i, 0)),
        out_shape=jax.ShapeDtypeStruct((M, H), _F32),
        compiler_params=pltpu.CompilerParams(
            dimension_semantics=("parallel",)),
    )(x2, h2b, pw, w1_t, b_fc.reshape(1, I), w2_t, b_proj.reshape(1, H))

    return out.reshape(B, S, H)


def kernel(hidden_states, ln1_s, ln1_b, W_qkv, b_qkv, W_o, b_o, ln2_s, ln2_b,
           W_r1, b_r1, W_r2, b_r2, W_fc, b_fc, W_proj, b_proj):
    return _run(hidden_states, ln1_s, ln1_b, W_qkv, b_qkv, W_o, b_o,
                ln2_s, ln2_b, W_r1, b_r1, W_r2, b_r2, W_fc, b_fc,
                W_proj, b_proj)


# speedup vs baseline: 1.4105x; 1.0509x over previous
import functools

import jax
import jax.numpy as jnp
from jax.experimental import pallas as pl
from jax.experimental.pallas import tpu as pltpu
from jax.experimental.pallas import tpu_sc as plsc

B, S, H = 2, 2048, 2048
NH = 16
DH = H // NH
P = 16
K = 4
RH = 256
I = 8192
HPP = H // P
IPP = I // P
M = B * S

BM_QKV = 512
BQ = 2048
BK = 512
BM_EPI = 512

_BF = jnp.bfloat16
_F32 = jnp.float32


def _dot_nt(a, b_t):
    return jax.lax.dot_general(
        a, b_t, (((1,), (1,)), ((), ())), preferred_element_type=_F32)


def _layer_norm(x, s, b, eps=1e-5):
    m = jnp.mean(x, axis=-1, keepdims=True)
    v = jnp.mean((x - m) ** 2, axis=-1, keepdims=True)
    return (x - m) * jax.lax.rsqrt(v + eps) * s + b


def _qkv_kernel(x_ref, s_ref, b_ref, w_ref, bias_ref, o_ref):
    h = _layer_norm(x_ref[...], s_ref[...], b_ref[...])
    acc = _dot_nt(h.astype(_BF), w_ref[...])
    o_ref[...] = (acc + bias_ref[...]).astype(_BF)


def _attn_kernel(q_ref, k_ref, v_ref, o_ref):
    qi = pl.program_id(2)
    q = q_ref[0]
    rq = BQ // BK

    def step(j, carry, masked):
        acc, m, l = carry
        k = k_ref[0, pl.ds(j * BK, BK), :]
        s = jax.lax.dot_general(
            q, k, (((1,), (1,)), ((), ())), preferred_element_type=_F32)
        if masked:
            qpos = qi * BQ + jax.lax.broadcasted_iota(
                jnp.int32, (BQ, BK), 0)
            kpos = j * BK + jax.lax.broadcasted_iota(
                jnp.int32, (BQ, BK), 1)
            s = jnp.where(qpos >= kpos, s, -1e30)
        m_new = jnp.maximum(m, jnp.max(s, axis=1, keepdims=True))
        p = jnp.exp((s - m_new).astype(_BF))
        alpha = jnp.exp(m - m_new)
        l = l * alpha + jnp.sum(p, axis=1, keepdims=True).astype(_F32)
        v = v_ref[0, pl.ds(j * BK, BK), :]
        acc = acc * alpha + jnp.dot(p, v, preferred_element_type=_F32)
        return acc, m_new, l

    acc0 = jnp.zeros((BQ, DH), _F32)
    m0 = jnp.full((BQ, 1), -1e30, _F32)
    l0 = jnp.zeros((BQ, 1), _F32)
    def pair(jj, c):
        return step(2 * jj + 1, step(2 * jj, c, False), False)

    carry = jax.lax.fori_loop(0, qi * rq // 2, pair, (acc0, m0, l0))
    for jj in range(rq):
        carry = step(qi * rq + jj, carry, True)
    acc, _, l = carry
    o_ref[0] = (acc / l).astype(_BF)


def _epi_a_kernel(res_ref, o_ref, wo_ref, bo_ref, s2_ref, b2_ref,
                  wr1_ref, br1_ref, wr2_ref, br2_ref,
                  x_ref, h2_ref, sc_ref):
    attn = _dot_nt(o_ref[...], wo_ref[...])
    x = res_ref[...] + attn + bo_ref[...]
    x_ref[...] = x
    h2 = _layer_norm(x, s2_ref[...], b2_ref[...])
    h2b = h2.astype(_BF)
    h2_ref[...] = h2b

    r = _dot_nt(h2b, wr1_ref[...]) + br1_ref[...]
    r = jnp.maximum(r, 0.0)
    sc_ref[...] = _dot_nt(r.astype(_BF), wr2_ref[...]) + br2_ref[...]


def _route_sc_kernel(nc, tokens_per_w, sc_hbm, out_hbm, in_v, out_v):
    wid = jax.lax.axis_index("s") * nc + jax.lax.axis_index("c")
    base = wid * tokens_per_w * P
    pltpu.sync_copy(sc_hbm.at[pl.ds(base, tokens_per_w * P)], in_v)
    iot = jax.lax.iota(jnp.int32, P)
    zero = jnp.zeros((P,), _F32)
    one = jnp.ones((P,), _F32)
    neg = jnp.full((P,), -1e30, _F32)

    def body(g, carry):
        idx0 = g * (P * P)
        idxs = [iot * P + (idx0 + p) for p in range(P)]
        s = [plsc.load_gather(in_v, [idxs[p]]) for p in range(P)]
        m = functools.reduce(jnp.maximum, s)
        e = [jnp.exp(sp - m) for sp in s]
        z = functools.reduce(jnp.add, e)
        cur = list(s)
        selected = [zero] * P
        for _ in range(K):
            cur_max = functools.reduce(jnp.maximum, cur)
            taken = zero
            for p in range(P):
                hit = jnp.where((cur[p] >= cur_max) & (taken < 0.5),
                                one, zero)
                selected[p] = selected[p] + hit
                taken = taken + hit
                cur[p] = jnp.where(hit > 0.5, neg, cur[p])
        w = [selected[p] * e[p] for p in range(P)]
        ssum = functools.reduce(jnp.add, w)
        inv = 1.0 / (ssum + 1e-8 * z)
        for p in range(P):
            plsc.store_scatter(out_v, [idxs[p]], w[p] * inv)
        return carry

    jax.lax.fori_loop(0, tokens_per_w // P, body, 0)
    pltpu.sync_copy(out_v, out_hbm.at[pl.ds(base, tokens_per_w * P)])


def _route_sc(scores_flat):
    info = plsc.get_sparse_core_info()
    nw = info.num_cores * info.num_subcores
    tokens_per_w = M // nw
    mesh = plsc.VectorSubcoreMesh(core_axis_name="c", subcore_axis_name="s")
    fn = pl.kernel(
        functools.partial(_route_sc_kernel, info.num_cores, tokens_per_w),
        out_type=jax.ShapeDtypeStruct((M * P,), _F32),
        mesh=mesh,
        scratch_types=[
            pltpu.VMEM((tokens_per_w * P,), _F32),
            pltpu.VMEM((tokens_per_w * P,), _F32),
        ],
        compiler_params=pltpu.CompilerParams(needs_layout_passes=False),
    )
    return fn(scores_flat)


def _epi_b_kernel(x_ref, h2_ref, pw_ref,
                  w1_ref, bfc_ref, w2_ref, bproj_ref, out_ref):
    x = x_ref[...]
    h2b = h2_ref[...]
    for i in range(P):
        hs = h2b[:, i * HPP:(i + 1) * HPP]
        w1 = w1_ref[pl.ds(i * IPP, IPP), :]
        inter = _dot_nt(hs, w1)
        inter = (inter + bfc_ref[:, i * IPP:(i + 1) * IPP]).astype(_BF)
        inter = jax.nn.gelu(inter)
        w2 = w2_ref[pl.ds(i * HPP, HPP), :]
        po = _dot_nt(inter, w2)
        po = po + bproj_ref[:, i * HPP:(i + 1) * HPP]
        out_ref[:, i * HPP:(i + 1) * HPP] = (
            x[:, i * HPP:(i + 1) * HPP] + po * pw_ref[:, i:i + 1])


@jax.jit
def _run(hidden_states, ln1_s, ln1_b, W_qkv, b_qkv, W_o, b_o, ln2_s, ln2_b,
         W_r1, b_r1, W_r2, b_r2, W_fc, b_fc, W_proj, b_proj):
    x = hidden_states.reshape(M, H)

    qk_scale = jnp.concatenate(
        [jnp.full((H, 1), 1.0 / (DH ** 0.5), _F32),
         jnp.ones((2 * H, 1), _F32)], axis=0)

    qkv = pl.pallas_call(
        _qkv_kernel,
        grid=(M // BM_QKV,),
        in_specs=[
            pl.BlockSpec((BM_QKV, H), lambda i: (i, 0)),
            pl.BlockSpec((1, H), lambda i: (0, 0)),
            pl.BlockSpec((1, H), lambda i: (0, 0)),
            pl.BlockSpec((3 * H, H), lambda i: (0, 0)),
            pl.BlockSpec((1, 3 * H), lambda i: (0, 0)),
        ],
        out_specs=pl.BlockSpec((BM_QKV, 3 * H), lambda i: (i, 0)),
        out_shape=jax.ShapeDtypeStruct((M, 3 * H), _BF),
        compiler_params=pltpu.CompilerParams(
            dimension_semantics=("parallel",)),
    )(x, ln1_s.reshape(1, H), ln1_b.reshape(1, H),
      (W_qkv * qk_scale).astype(_BF), (b_qkv * qk_scale.reshape(3 * H))
      .reshape(1, 3 * H))

    qkv3 = qkv.reshape(B, S, 3 * H)

    o = pl.pallas_call(
        _attn_kernel,
        grid=(B, NH, S // BQ),
        in_specs=[
            pl.BlockSpec((1, BQ, DH), lambda b, h, i: (b, i, h)),
            pl.BlockSpec((1, S, DH), lambda b, h, i: (b, 0, NH + h)),
            pl.BlockSpec((1, S, DH), lambda b, h, i: (b, 0, 2 * NH + h)),
        ],
        out_specs=pl.BlockSpec((1, BQ, DH), lambda b, h, i: (b, i, h)),
        out_shape=jax.ShapeDtypeStruct((B, S, H), _BF),
        compiler_params=pltpu.CompilerParams(
            dimension_semantics=("parallel", "parallel", "arbitrary")),
    )(qkv3, qkv3, qkv3)

    o2 = o.reshape(M, H)

    w1_t = jnp.concatenate(
        [W_fc[i * IPP:(i + 1) * IPP, i * HPP:(i + 1) * HPP]
         for i in range(P)], axis=0).astype(_BF)
    w2_t = jnp.concatenate(
        [W_proj[i * HPP:(i + 1) * HPP, i * IPP:(i + 1) * IPP]
         for i in range(P)], axis=0).astype(_BF)

    x2, h2b, scores = pl.pallas_call(
        _epi_a_kernel,
        grid=(M // BM_EPI,),
        in_specs=[
            pl.BlockSpec((BM_EPI, H), lambda i: (i, 0)),
            pl.BlockSpec((BM_EPI, H), lambda i: (i, 0)),
            pl.BlockSpec((H, H), lambda i: (0, 0)),
            pl.BlockSpec((1, H), lambda i: (0, 0)),
            pl.BlockSpec((1, H), lambda i: (0, 0)),
            pl.BlockSpec((1, H), lambda i: (0, 0)),
            pl.BlockSpec((RH, H), lambda i: (0, 0)),
            pl.BlockSpec((1, RH), lambda i: (0, 0)),
            pl.BlockSpec((P, RH), lambda i: (0, 0)),
            pl.BlockSpec((1, P), lambda i: (0, 0)),
        ],
        out_specs=[
            pl.BlockSpec((BM_EPI, H), lambda i: (i, 0)),
            pl.BlockSpec((BM_EPI, H), lambda i: (i, 0)),
            pl.BlockSpec((BM_EPI, P), lambda i: (i, 0)),
        ],
        out_shape=[
            jax.ShapeDtypeStruct((M, H), _F32),
            jax.ShapeDtypeStruct((M, H), _BF),
            jax.ShapeDtypeStruct((M, P), _F32),
        ],
        compiler_params=pltpu.CompilerParams(
            dimension_semantics=("parallel",)),
    )(x, o2, W_o.astype(_BF), b_o.reshape(1, H),
      ln2_s.reshape(1, H), ln2_b.reshape(1, H),
      W_r1.astype(_BF), b_r1.reshape(1, RH),
      W_r2.astype(_BF), b_r2.reshape(1, P))

    pw = _route_sc(scores.reshape(M * P)).reshape(M, P)

    out = pl.pallas_call(
        _epi_b_kernel,
        grid=(M // BM_EPI,),
        in_specs=[
            pl.BlockSpec((BM_EPI, H), lambda i: (i, 0)),
            pl.BlockSpec((BM_EPI, H), lambda i: (i, 0)),
            pl.BlockSpec((BM_EPI, P), lambda i: (i, 0)),
            pl.BlockSpec((I, HPP), lambda i: (0, 0)),
            pl.BlockSpec((1, I), lambda i: (0, 0)),
            pl.BlockSpec((H, IPP), lambda i: (0, 0)),
            pl.BlockSpec((1, H), lambda i: (0, 0)),
        ],
        out_specs=pl.BlockSpec((BM_EPI, H), lambda i: (i, 0)),
        out_shape=jax.ShapeDtypeStruct((M, H), _F32),
        compiler_params=pltpu.CompilerParams(
            dimension_semantics=("parallel",)),
    )(x2, h2b, pw, w1_t, b_fc.reshape(1, I), w2_t, b_proj.reshape(1, H))

    return out.reshape(B, S, H)


def kernel(hidden_states, ln1_s, ln1_b, W_qkv, b_qkv, W_o, b_o, ln2_s, ln2_b,
           W_r1, b_r1, W_r2, b_r2, W_fc, b_fc, W_proj, b_proj):
    return _run(hidden_states, ln1_s, ln1_b, W_qkv, b_qkv, W_o, b_o,
                ln2_s, ln2_b, W_r1, b_r1, W_r2, b_r2, W_fc, b_fc,
                W_proj, b_proj)
